# baseline (speedup 1.0000x reference)
import functools

import jax
import jax.numpy as jnp
from jax import lax
from jax.experimental import pallas as pl
from jax.experimental.pallas import tpu as pltpu
from jax.experimental.pallas import tpu_sc as plsc

ROWS = 128
COLS = 100000
NW = 32
ROWS_PER_W = ROWS // NW
CHUNK = 20000
NCHUNK = COLS // CHUNK
U = 5
GROUP = U * 16
ITERS = CHUNK // GROUP
NEG_INF = float("-inf")
BIG = 2**31 - 1


def _chunk_scan(buf, i0, carry):
    vms, vits = carry

    def body(i, c):
        vms, vits = c
        off = (i - i0) * GROUP
        ib = jnp.full((16,), 0, jnp.int32) + i
        new_vms = []
        new_vits = []
        for u in range(U):
            v = buf[pl.ds(off + u * 16, 16)]
            pred = v > vms[u]
            new_vits.append(jnp.where(pred, ib, vits[u]))
            new_vms.append(jnp.maximum(vms[u], v))
        return tuple(new_vms), tuple(new_vits)

    return lax.fori_loop(i0, i0 + ITERS, body, (vms, vits), unroll=2)


def _row_result(carry, lane_iota, perms):
    vms, vits = carry
    m = vms[0]
    for u in range(1, U):
        m = jnp.maximum(m, vms[u])
    for perm in perms:
        m = jnp.maximum(m, m[perm])
    cand = None
    for u in range(U):
        full_idx = vits[u] * GROUP + (u * 16) + lane_iota
        c = jnp.where(vms[u] == m, full_idx, jnp.int32(BIG))
        cand = c if cand is None else jnp.minimum(cand, c)
    for perm in perms:
        cand = jnp.minimum(cand, cand[perm])
    return cand


def _greedy_body(x_hbm, out_hbm, buf0, buf1, res_v, sem0, sem1):
    wid = lax.axis_index("s") * 2 + lax.axis_index("c")
    bufs = (buf0, buf1)
    sems = (sem0, sem1)
    lane_iota = lax.iota(jnp.int32, 16)
    perms = [jnp.bitwise_xor(lane_iota, sh) for sh in (8, 4, 2, 1)]
    total = ROWS_PER_W * NCHUNK

    def start(t):
        j, c = divmod(t, NCHUNK)
        off = ((ROWS_PER_W * wid + j) * COLS + c * CHUNK)
        off = pl.multiple_of(off, 32)
        d = pltpu.make_async_copy(
            x_hbm.at[pl.ds(off, CHUNK)], bufs[t % 2], sems[t % 2]
        )
        d.start()
        return d

    handles = [start(0), start(1)]
    fresh = (
        tuple(jnp.full((16,), NEG_INF, jnp.float32) for _ in range(U)),
        tuple(jnp.full((16,), 0, jnp.int32) for _ in range(U)),
    )
    carry = fresh
    results = jnp.full((16,), 0, jnp.int32)
    for t in range(total):
        j, c = divmod(t, NCHUNK)
        handles[t % 2].wait()
        carry = _chunk_scan(bufs[t % 2], c * ITERS, carry)
        if t + 2 < total:
            handles[t % 2] = start(t + 2)
        if c == NCHUNK - 1:
            idx = _row_result(carry, lane_iota, perms)
            results = jnp.where(lane_iota == j, idx, results)
            carry = fresh
    res_v[...] = results
    pltpu.sync_copy(res_v, out_hbm.at[wid])


@jax.jit
def kernel(m_logits):
    x = m_logits.reshape(-1)
    mesh = plsc.VectorSubcoreMesh(core_axis_name="c", subcore_axis_name="s")
    out = pl.kernel(
        _greedy_body,
        out_type=jax.ShapeDtypeStruct((NW, 16), jnp.int32),
        mesh=mesh,
        scratch_types=[
            pltpu.VMEM((CHUNK,), jnp.float32),
            pltpu.VMEM((CHUNK,), jnp.float32),
            pltpu.VMEM((16,), jnp.int32),
            pltpu.SemaphoreType.DMA,
            pltpu.SemaphoreType.DMA,
        ],
    )(x)
    return out[:, :ROWS_PER_W].reshape(ROWS, 1).astype(jnp.int64)


# speedup vs baseline: 1.7508x; 1.7508x over previous
import functools

import jax
import jax.numpy as jnp
from jax import lax
from jax.experimental import pallas as pl
from jax.experimental.pallas import tpu as pltpu
from jax.experimental.pallas import tpu_sc as plsc

ROWS = 128
COLS = 100000
COLS_PAD = 100096
TILE_R = 8
NTR = ROWS // TILE_R
HALF_TILES = 391
HALF_COLS = HALF_TILES * 128
BLK_TILES = 23
BLK_COLS = BLK_TILES * 128
NBLK = HALF_TILES // BLK_TILES
NV = BLK_COLS // 16
NV_VALID_LAST = 178
BIG = 2**31 - 1
NEG_INF = float("-inf")


def _scan_block(buf, base, n0, n_masked, p_is_odd, carry):
    vms, vits = carry

    def step(i, c, masked):
        vms, vits = c
        ib = jnp.broadcast_to(i + base, (16,))
        off = i * 16
        new_vms, new_vits = [], []
        for j in range(TILE_R):
            v = buf[j, pl.ds(off, 16)]
            if masked:
                v = jnp.minimum(v, p_is_odd)
            pred = v > vms[j]
            new_vits.append(jnp.where(pred, ib, vits[j]))
            new_vms.append(jnp.maximum(vms[j], v))
        return tuple(new_vms), tuple(new_vits)

    carry = lax.fori_loop(
        0, n0, lambda i, c: step(i, c, False), (vms, vits), unroll=2
    )
    for i in range(n0, n0 + n_masked):
        carry = step(i, carry, True)
    return carry


def _reduce_rows(carry, lane_iota, perms, col_base):
    vms, vits = carry
    vals, idxs = [], []
    for j in range(TILE_R):
        m = vms[j]
        for perm in perms:
            m = jnp.maximum(m, m[perm])
        full_idx = col_base + vits[j] * 16 + lane_iota
        cand = jnp.where(vms[j] == m, full_idx, jnp.int32(BIG))
        for perm in perms:
            cand = jnp.minimum(cand, cand[perm])
        vals.append(m)
        idxs.append(cand)
    return vals, idxs


def _greedy_body(x_hbm, out_hbm, buf0, buf1, myval_v, myidx_v, pval_v,
                 pidx_v, res_v, shval, shidx, sem0, sem1):
    cid = lax.axis_index("c")
    sid = lax.axis_index("s")
    p = sid % 2
    tr = cid * 8 + sid // 2
    row0 = pl.multiple_of(tr * TILE_R, TILE_R)
    col_base = p * HALF_COLS
    bufs = (buf0, buf1)
    sems = (sem0, sem1)
    lane_iota = lax.iota(jnp.int32, 16)
    perms = [jnp.bitwise_xor(lane_iota, sh) for sh in (8, 4, 2, 1)]
    limit = jnp.where(p == 1, jnp.float32(NEG_INF), jnp.float32(float("inf")))
    p_is_odd = jnp.broadcast_to(limit, (16,))

    def start(t):
        col0 = pl.multiple_of(col_base + t * BLK_COLS, 128)
        d = pltpu.make_async_copy(
            x_hbm.at[pl.ds(row0, TILE_R), pl.ds(col0, BLK_COLS)],
            bufs[t % 2], sems[t % 2],
        )
        d.start()
        return d

    handles = [start(0), start(1)]
    carry = (
        tuple(jnp.full((16,), NEG_INF, jnp.float32) for _ in range(TILE_R)),
        tuple(jnp.full((16,), 0, jnp.int32) for _ in range(TILE_R)),
    )
    for t in range(NBLK):
        handles[t % 2].wait()
        if t == NBLK - 1:
            carry = _scan_block(
                bufs[t % 2], t * NV, NV_VALID_LAST, NV - NV_VALID_LAST,
                p_is_odd, carry,
            )
        else:
            carry = _scan_block(bufs[t % 2], t * NV, NV, 0, p_is_odd, carry)
        if t + 2 < NBLK:
            handles[t % 2] = start(t + 2)

    vals, idxs = _reduce_rows(carry, lane_iota, perms, col_base)
    myval = jnp.full((16,), NEG_INF, jnp.float32)
    myidx = jnp.full((16,), 0, jnp.int32)
    for j in range(TILE_R):
        sel = lane_iota == j
        myval = jnp.where(sel, vals[j], myval)
        myidx = jnp.where(sel, idxs[j], myidx)
    for k in range(8):
        myval_v[pl.ds(k * 16, 16)] = myval
        myidx_v[pl.ds(k * 16, 16)] = myidx
    pltpu.sync_copy(myval_v, shval.at[sid])
    pltpu.sync_copy(myidx_v, shidx.at[sid])
    plsc.subcore_barrier()

    @pl.when(p == 0)
    def _():
        pltpu.sync_copy(shval.at[sid + 1], pval_v)
        pltpu.sync_copy(shidx.at[sid + 1], pidx_v)
        pv = pval_v[pl.ds(0, 16)]
        pi = pidx_v[pl.ds(0, 16)]
        take = pv > myval
        res_v[...] = jnp.where(take, pi, myidx)
        pltpu.sync_copy(res_v, out_hbm.at[tr])


@jax.jit
def kernel(m_logits):
    mesh = plsc.VectorSubcoreMesh(core_axis_name="c", subcore_axis_name="s")
    out = pl.kernel(
        _greedy_body,
        out_type=jax.ShapeDtypeStruct((NTR, 16), jnp.int32),
        mesh=mesh,
        scratch_types=[
            pltpu.VMEM((TILE_R, BLK_COLS), jnp.float32),
            pltpu.VMEM((TILE_R, BLK_COLS), jnp.float32),
            pltpu.VMEM((128,), jnp.float32),
            pltpu.VMEM((128,), jnp.int32),
            pltpu.VMEM((128,), jnp.float32),
            pltpu.VMEM((128,), jnp.int32),
            pltpu.VMEM((16,), jnp.int32),
            pltpu.VMEM_SHARED((16, 128), jnp.float32),
            pltpu.VMEM_SHARED((16, 128), jnp.int32),
            pltpu.SemaphoreType.DMA,
            pltpu.SemaphoreType.DMA,
        ],
    )(m_logits)
    return out[:, :TILE_R].reshape(ROWS, 1).astype(jnp.int64)


# speedup vs baseline: 3.0591x; 1.7472x over previous
import functools

import jax
import jax.numpy as jnp
from jax import lax
from jax.experimental import pallas as pl
from jax.experimental.pallas import tpu as pltpu
from jax.experimental.pallas import tpu_sc as plsc

ROWS = 128
COLS = 100000
NW = 32
SPAN = 3128
BLK = 184
NBLK = SPAN // BLK
NACC = ROWS // 16
LAST_START = COLS - BLK
NEG_INF = float("-inf")


def _scan_block(buf, r0, carry):
    vms, vits = carry

    def step(i, c):
        vms, vits = c
        ib = jnp.broadcast_to(r0 + i, (16,))
        nm, ni = [], []
        for k in range(NACC):
            v = buf[i, pl.ds(k * 16, 16)]
            pred = v > vms[k]
            ni.append(jnp.where(pred, ib, vits[k]))
            nm.append(jnp.maximum(vms[k], v))
        return tuple(nm), tuple(ni)

    return lax.fori_loop(0, BLK, step, (vms, vits), unroll=2)


def _greedy_body(xt_hbm, outi_hbm, outv_hbm, buf0, buf1, stage_v, stage_i,
                 pval_v, pidx_v, shval, shidx, sem0, sem1):
    cid = lax.axis_index("c")
    sid = lax.axis_index("s")
    w = cid * 16 + sid
    r0 = w * SPAN
    bufs = (buf0, buf1)
    sems = (sem0, sem1)

    def start(t):
        st = jnp.minimum(r0 + t * BLK, LAST_START)
        st = pl.multiple_of(st, 8)
        d = pltpu.make_async_copy(
            xt_hbm.at[pl.ds(st, BLK), :], bufs[t % 2], sems[t % 2]
        )
        d.start()
        return d, st

    handles = [start(0), start(1)]
    carry = (
        tuple(jnp.full((16,), NEG_INF, jnp.float32) for _ in range(NACC)),
        tuple(jnp.full((16,), 0, jnp.int32) for _ in range(NACC)),
    )
    for t in range(NBLK):
        d, st = handles[t % 2]
        d.wait()
        carry = _scan_block(bufs[t % 2], st, carry)
        if t + 2 < NBLK:
            handles[t % 2] = start(t + 2)

    vms, vits = carry
    for k in range(NACC):
        stage_v[pl.ds(k * 16, 16)] = vms[k]
        stage_i[pl.ds(k * 16, 16)] = vits[k]
    pltpu.sync_copy(stage_v, shval.at[sid])
    pltpu.sync_copy(stage_i, shidx.at[sid])
    plsc.subcore_barrier()

    @pl.when(sid == 0)
    def _():
        accs = list(vms)
        acci = list(vits)
        for s in range(1, 16):
            pltpu.sync_copy(shval.at[s], pval_v)
            pltpu.sync_copy(shidx.at[s], pidx_v)
            for k in range(NACC):
                pv = pval_v[pl.ds(k * 16, 16)]
                pi = pidx_v[pl.ds(k * 16, 16)]
                take = pv > accs[k]
                acci[k] = jnp.where(take, pi, acci[k])
                accs[k] = jnp.maximum(accs[k], pv)
        for k in range(NACC):
            stage_v[pl.ds(k * 16, 16)] = accs[k]
            stage_i[pl.ds(k * 16, 16)] = acci[k]
        pltpu.sync_copy(stage_i, outi_hbm.at[cid])
        pltpu.sync_copy(stage_v, outv_hbm.at[cid])


@jax.jit
def kernel(m_logits):
    xt = m_logits.T
    mesh = plsc.VectorSubcoreMesh(core_axis_name="c", subcore_axis_name="s")
    idx2, val2 = pl.kernel(
        _greedy_body,
        out_type=(
            jax.ShapeDtypeStruct((2, 128), jnp.int32),
            jax.ShapeDtypeStruct((2, 128), jnp.float32),
        ),
        mesh=mesh,
        scratch_types=[
            pltpu.VMEM((BLK, 128), jnp.float32),
            pltpu.VMEM((BLK, 128), jnp.float32),
            pltpu.VMEM((128,), jnp.float32),
            pltpu.VMEM((128,), jnp.int32),
            pltpu.VMEM((128,), jnp.float32),
            pltpu.VMEM((128,), jnp.int32),
            pltpu.VMEM_SHARED((16, 128), jnp.float32),
            pltpu.VMEM_SHARED((16, 128), jnp.int32),
            pltpu.SemaphoreType.DMA,
            pltpu.SemaphoreType.DMA,
        ],
    )(xt)
    token = jnp.where(val2[1] > val2[0], idx2[1], idx2[0])
    return token.reshape(ROWS, 1).astype(jnp.int64)


# speedup vs baseline: 3.3174x; 1.0844x over previous
import functools

import jax
import jax.numpy as jnp
from jax import lax
from jax.experimental import pallas as pl
from jax.experimental.pallas import tpu as pltpu
from jax.experimental.pallas import tpu_sc as plsc

ROWS = 128
COLS = 100000
NW = 32
SPAN = 3136
BLK = 392
NBLK = SPAN // BLK
NACC = ROWS // 16
LAST_START = COLS - BLK
NEG_INF = float("-inf")


def _scan_block(buf, r0, carry):
    vms, vits = carry

    def step(i, c):
        vms, vits = c
        ib = jnp.broadcast_to(r0 + i, (16,))
        nm, ni = [], []
        for k in range(NACC):
            v = buf[i, pl.ds(k * 16, 16)]
            pred = v > vms[k]
            ni.append(jnp.where(pred, ib, vits[k]))
            nm.append(jnp.maximum(vms[k], v))
        return tuple(nm), tuple(ni)

    return lax.fori_loop(0, BLK, step, (vms, vits), unroll=2)


def _greedy_body(xt_hbm, outi_hbm, outv_hbm, buf0, buf1, stage_v, stage_i,
                 pval_v, pidx_v, shval, shidx, sem0, sem1):
    cid = lax.axis_index("c")
    sid = lax.axis_index("s")
    w = cid * 16 + sid
    r0 = w * SPAN
    bufs = (buf0, buf1)
    sems = (sem0, sem1)

    def start(t):
        st = jnp.minimum(r0 + t * BLK, LAST_START)
        st = pl.multiple_of(st, 8)
        d = pltpu.make_async_copy(
            xt_hbm.at[pl.ds(st, BLK), :], bufs[t % 2], sems[t % 2]
        )
        d.start()
        return d, st

    handles = [start(0), start(1)]
    carry = (
        tuple(jnp.full((16,), NEG_INF, jnp.float32) for _ in range(NACC)),
        tuple(jnp.full((16,), 0, jnp.int32) for _ in range(NACC)),
    )
    for t in range(NBLK):
        d, st = handles[t % 2]
        d.wait()
        carry = _scan_block(bufs[t % 2], st, carry)
        if t + 2 < NBLK:
            handles[t % 2] = start(t + 2)

    vms, vits = carry
    for k in range(NACC):
        stage_v[pl.ds(k * 16, 16)] = vms[k]
        stage_i[pl.ds(k * 16, 16)] = vits[k]
    pltpu.sync_copy(stage_v, shval.at[sid])
    pltpu.sync_copy(stage_i, shidx.at[sid])
    plsc.subcore_barrier()

    @pl.when(sid == 0)
    def _():
        def fold(s, c):
            accs, acci = c
            pltpu.sync_copy(shval.at[s], pval_v)
            pltpu.sync_copy(shidx.at[s], pidx_v)
            na, ni = [], []
            for k in range(NACC):
                pv = pval_v[pl.ds(k * 16, 16)]
                pi = pidx_v[pl.ds(k * 16, 16)]
                take = pv > accs[k]
                ni.append(jnp.where(take, pi, acci[k]))
                na.append(jnp.maximum(accs[k], pv))
            return tuple(na), tuple(ni)

        accs, acci = lax.fori_loop(1, 16, fold, (vms, vits))
        for k in range(NACC):
            stage_v[pl.ds(k * 16, 16)] = accs[k]
            stage_i[pl.ds(k * 16, 16)] = acci[k]
        pltpu.sync_copy(stage_i, outi_hbm.at[cid])
        pltpu.sync_copy(stage_v, outv_hbm.at[cid])


@jax.jit
def kernel(m_logits):
    xt = m_logits.T
    mesh = plsc.VectorSubcoreMesh(core_axis_name="c", subcore_axis_name="s")
    idx2, val2 = pl.kernel(
        _greedy_body,
        out_type=(
            jax.ShapeDtypeStruct((2, 128), jnp.int32),
            jax.ShapeDtypeStruct((2, 128), jnp.float32),
        ),
        mesh=mesh,
        scratch_types=[
            pltpu.VMEM((BLK, 128), jnp.float32),
            pltpu.VMEM((BLK, 128), jnp.float32),
            pltpu.VMEM((128,), jnp.float32),
            pltpu.VMEM((128,), jnp.int32),
            pltpu.VMEM((128,), jnp.float32),
            pltpu.VMEM((128,), jnp.int32),
            pltpu.VMEM_SHARED((16, 128), jnp.float32),
            pltpu.VMEM_SHARED((16, 128), jnp.int32),
            pltpu.SemaphoreType.DMA,
            pltpu.SemaphoreType.DMA,
        ],
    )(xt)
    token = jnp.where(val2[1] > val2[0], idx2[1], idx2[0])
    return token.reshape(ROWS, 1).astype(jnp.int64)


# speedup vs baseline: 3.3565x; 1.0118x over previous
import functools

import jax
import jax.numpy as jnp
from jax import lax
from jax.experimental import pallas as pl
from jax.experimental.pallas import tpu as pltpu
from jax.experimental.pallas import tpu_sc as plsc

ROWS = 128
COLS = 100000
NW = 32
SPAN = 3136
BLK = 392
NBLK = SPAN // BLK
NACC = ROWS // 16
LAST_START = COLS - BLK
NEG_INF = float("-inf")


def _scan_block(buf, r0, carry):
    vms, vits = carry

    def step(i, c):
        vms, vits = c
        ib = jnp.broadcast_to(r0 + i, (16,))
        nm, ni = [], []
        for k in range(NACC):
            v = buf[i, pl.ds(k * 16, 16)]
            pred = v > vms[k]
            ni.append(jnp.where(pred, ib, vits[k]))
            nm.append(jnp.maximum(vms[k], v))
        return tuple(nm), tuple(ni)

    return lax.fori_loop(0, BLK, step, (vms, vits), unroll=2)


def _greedy_body(xt_hbm, outi_hbm, outv_hbm, buf0, buf1, stage_v, stage_i,
                 pval_v, pidx_v, shval, shidx, sem0, sem1):
    cid = lax.axis_index("c")
    sid = lax.axis_index("s")
    w = cid * 16 + sid
    r0 = w * SPAN
    bufs = (buf0, buf1)
    sems = (sem0, sem1)

    def blk_start(t):
        return pl.multiple_of(jnp.minimum(r0 + t * BLK, LAST_START), 8)

    def start(t, b):
        pltpu.make_async_copy(
            xt_hbm.at[pl.ds(blk_start(t), BLK), :], bufs[b], sems[b]
        ).start()

    def wait(b):
        pltpu.make_async_copy(
            xt_hbm.at[pl.ds(pl.multiple_of(r0, 8), BLK), :], bufs[b], sems[b]
        ).wait()

    start(0, 0)
    start(1, 1)
    init = (
        tuple(jnp.full((16,), NEG_INF, jnp.float32) for _ in range(NACC)),
        tuple(jnp.full((16,), 0, jnp.int32) for _ in range(NACC)),
    )

    def pair(tp, carry):
        for b in range(2):
            t = tp * 2 + b
            wait(b)
            carry = _scan_block(bufs[b], blk_start(t), carry)

            @pl.when(t + 2 < NBLK)
            def _():
                start(t + 2, b)

        return carry

    vms, vits = lax.fori_loop(0, NBLK // 2, pair, init)
    for k in range(NACC):
        stage_v[pl.ds(k * 16, 16)] = vms[k]
        stage_i[pl.ds(k * 16, 16)] = vits[k]
    pltpu.sync_copy(stage_v, shval.at[sid])
    pltpu.sync_copy(stage_i, shidx.at[sid])
    plsc.subcore_barrier()

    @pl.when(sid == 0)
    def _():
        def fold(s, c):
            accs, acci = c
            pltpu.sync_copy(shval.at[s], pval_v)
            pltpu.sync_copy(shidx.at[s], pidx_v)
            na, ni = [], []
            for k in range(NACC):
                pv = pval_v[pl.ds(k * 16, 16)]
                pi = pidx_v[pl.ds(k * 16, 16)]
                take = pv > accs[k]
                ni.append(jnp.where(take, pi, acci[k]))
                na.append(jnp.maximum(accs[k], pv))
            return tuple(na), tuple(ni)

        accs, acci = lax.fori_loop(1, 16, fold, (vms, vits))
        for k in range(NACC):
            stage_v[pl.ds(k * 16, 16)] = accs[k]
            stage_i[pl.ds(k * 16, 16)] = acci[k]
        pltpu.sync_copy(stage_i, outi_hbm.at[cid])
        pltpu.sync_copy(stage_v, outv_hbm.at[cid])


@jax.jit
def kernel(m_logits):
    xt = m_logits.T
    mesh = plsc.VectorSubcoreMesh(core_axis_name="c", subcore_axis_name="s")
    idx2, val2 = pl.kernel(
        _greedy_body,
        out_type=(
            jax.ShapeDtypeStruct((2, 128), jnp.int32),
            jax.ShapeDtypeStruct((2, 128), jnp.float32),
        ),
        mesh=mesh,
        scratch_types=[
            pltpu.VMEM((BLK, 128), jnp.float32),
            pltpu.VMEM((BLK, 128), jnp.float32),
            pltpu.VMEM((128,), jnp.float32),
            pltpu.VMEM((128,), jnp.int32),
            pltpu.VMEM((128,), jnp.float32),
            pltpu.VMEM((128,), jnp.int32),
            pltpu.VMEM_SHARED((16, 128), jnp.float32),
            pltpu.VMEM_SHARED((16, 128), jnp.int32),
            pltpu.SemaphoreType.DMA,
            pltpu.SemaphoreType.DMA,
        ],
    )(xt)
    token = jnp.where(val2[1] > val2[0], idx2[1], idx2[0])
    return token.reshape(ROWS, 1).astype(jnp.int64)
